# Initial kernel scaffold; baseline (speedup 1.0000x reference)
import jax
import jax.numpy as jnp
from jax.experimental import pallas as pl


def kernel(X1, X2, sub_u_idx, sub_v_idx, bdu_u_idx, bdu_v_idx, bdc_u_idx, bdc_v_idx, Wenc_g, benc_g, Wconv_g, bconv_g, Wenc_s, benc_s, Wconv_s, bconv_s, Wenc_u, benc_u, Wconv_u, bconv_u, Wenc_c, benc_c, Wconv_c, bconv_c, Wro, bro, Wf0, bf0, Wf1, bf1, Wf2, bf2, Wf3, bf3, Wf4, bf4, Wf5, bf5):
    raise NotImplementedError("write your pallas kernel here")


# speedup vs baseline: 1.0656x; 1.0656x over previous
import functools

import jax
import jax.numpy as jnp
from jax import lax
from jax.experimental import pallas as pl
from jax.experimental.pallas import tpu as pltpu
from jax.experimental.pallas import tpu_sc as plsc

_D = 256
_N = 10000
_NROWS = 80
_LANES = 16


def _counts_sc(idx_all):
    mesh = plsc.VectorSubcoreMesh(core_axis_name="c", subcore_axis_name="s")

    @functools.partial(
        pl.kernel,
        out_type=jax.ShapeDtypeStruct((_NROWS, _N), jnp.float32),
        mesh=mesh,
        compiler_params=pltpu.CompilerParams(needs_layout_passes=False),
        scratch_types=[
            pltpu.VMEM((_N,), jnp.float32),
            pltpu.VMEM((512,), jnp.int32),
        ],
    )
    def counts_kernel(idx_hbm, out_hbm, rowbuf, idxbuf):
        wid = lax.axis_index("s") * 2 + lax.axis_index("c")
        ones16 = jnp.full((_LANES,), 1.0, jnp.float32)
        zeros16 = jnp.zeros((_LANES,), jnp.float32)
        lane = lax.iota(jnp.int32, _LANES)

        def fill(val):
            def body(i, c):
                rowbuf[pl.ds(i * _LANES, _LANES)] = jnp.full(
                    (_LANES,), val, jnp.float32)
                return c
            lax.fori_loop(0, _N // _LANES, body, 0)

        def load_idx(start, count):
            pltpu.sync_copy(idx_hbm.at[pl.ds(start, count)],
                            idxbuf.at[pl.ds(0, count)])

        def scatter(count):
            def body(i, c):
                iv = idxbuf[pl.ds(i * _LANES, _LANES)]
                for l in range(_LANES):
                    plsc.addupdate_scatter(rowbuf, [iv], ones16,
                                           mask=lane == l)
                return c
            lax.fori_loop(0, count // _LANES, body, 0)

        def unscatter(count):
            def body(i, c):
                iv = idxbuf[pl.ds(i * _LANES, _LANES)]
                plsc.store_scatter(rowbuf, [iv], zeros16)
                return c
            lax.fori_loop(0, count // _LANES, body, 0)

        def emit(r):
            pltpu.sync_copy(rowbuf, out_hbm.at[r])

        def do_row(r, start, count):
            load_idx(start, count)
            scatter(count)
            emit(r)
            unscatter(count)

        fill(0.0)
        for off in (0, 32):
            do_row(wid + off, 4608 + 128 * (wid + off), 128)

        for p in range(4):
            @pl.when(wid == 16 + p)
            def _(p=p):
                do_row(64 + p, 128 * p, 128)

        for p in range(8):
            @pl.when(wid == 20 + p)
            def _(p=p):
                do_row(68 + p, 512 + 512 * p, 512)

        @pl.when(wid == 28)
        def _():
            fill(1.0)
            emit(76)

        for p, w in ((77, 29), (78, 30), (79, 31)):
            @pl.when(wid == w)
            def _(p=p):
                emit(p)

    return counts_kernel(idx_all)


def _mm(a, b):
    return lax.dot_general(a, b, (((1,), (0,)), ((), ())),
                           precision=lax.Precision.HIGHEST,
                           preferred_element_type=jnp.float32)


def _dense_body(x1_ref, c_ref,
                wc_ref, bc_ref, Bc_ref, bbc_ref,
                ws_ref, bs_ref, Bs_ref, bbs_ref,
                wu_ref, bu_ref, Bu_ref, bbu_ref,
                wg_ref, bg_ref, Bg_ref, bbg_ref,
                wro_ref, bro_ref,
                wf0_ref, bf0_ref, wf1_ref, bf1_ref, wf2_ref, bf2_ref,
                wf3_ref, bf3_ref, wf4_ref, bf4_ref, wf5_ref, bf5_ref,
                out_ref):
    C = c_ref[...]
    X = x1_ref[...]
    acc = _mm(C, X)

    Zc = acc[0:64]
    Zx = acc[64:80]
    rows16 = lax.broadcasted_iota(jnp.int32, (16, 1), 0)
    Zs = jnp.sum(jnp.where(rows16 < 4, Zx, 0.0), axis=0, keepdims=True)
    Zu = jnp.sum(jnp.where((rows16 >= 4) & (rows16 < 12), Zx, 0.0),
                 axis=0, keepdims=True)
    Zg = jnp.sum(jnp.where(rows16 == 12, Zx, 0.0), axis=0, keepdims=True)

    def enc(Z, w_ref, b_ref, B_ref, bb_ref):
        A = jnp.maximum(_mm(Z, w_ref[...]) + b_ref[...], 0.0)
        return jnp.tanh(2.0 * (_mm(A, B_ref[...]) + bb_ref[...]))

    Hc = enc(Zc, wc_ref, bc_ref, Bc_ref, bbc_ref)
    HS = enc(Zs, ws_ref, bs_ref, Bs_ref, bbs_ref)
    HBDU = enc(Zu, wu_ref, bu_ref, Bu_ref, bbu_ref)
    HG = enc(Zg, wg_ref, bg_ref, Bg_ref, bbg_ref)

    S = jnp.sum(Hc, axis=0, keepdims=True)
    HBDC = jnp.maximum(_mm(S, wro_ref[...]) + bro_ref[...], 0.0)

    h = jnp.concatenate([HG, HS, HBDU, HBDC], axis=1)
    for w_ref, b_ref in ((wf0_ref, bf0_ref), (wf1_ref, bf1_ref),
                         (wf2_ref, bf2_ref), (wf3_ref, bf3_ref),
                         (wf4_ref, bf4_ref)):
        z = _mm(h, w_ref[...]) + b_ref[...]
        h = jnp.where(z > 0.0, z, jnp.exp(z) - 1.0)
    q = _mm(h, wf5_ref[...]) + bf5_ref[...]
    out_ref[...] = q


def _conv_as_matrix(Wconv, bconv):
    B = (jnp.eye(_LANES, dtype=jnp.float32)[:, None, None, :]
         * Wconv.T[None, :, :, None])
    B = B.reshape(_D, 128)
    bb = jnp.repeat(bconv, _LANES).reshape(1, 128)
    return B, bb


def kernel(X1, X2, sub_u_idx, sub_v_idx, bdu_u_idx, bdu_v_idx,
           bdc_u_idx, bdc_v_idx,
           Wenc_g, benc_g, Wconv_g, bconv_g,
           Wenc_s, benc_s, Wconv_s, bconv_s,
           Wenc_u, benc_u, Wconv_u, bconv_u,
           Wenc_c, benc_c, Wconv_c, bconv_c,
           Wro, bro, Wf0, bf0, Wf1, bf1, Wf2, bf2, Wf3, bf3, Wf4, bf4,
           Wf5, bf5):
    idx_all = jnp.concatenate([
        sub_u_idx.astype(jnp.int32),
        bdu_u_idx.astype(jnp.int32),
        bdc_u_idx.reshape(-1).astype(jnp.int32),
    ])
    counts = _counts_sc(idx_all)

    Bc, bbc = _conv_as_matrix(Wconv_c, bconv_c)
    Bs, bbs = _conv_as_matrix(Wconv_s, bconv_s)
    Bu, bbu = _conv_as_matrix(Wconv_u, bconv_u)
    Bg, bbg = _conv_as_matrix(Wconv_g, bconv_g)

    out = pl.pallas_call(
        _dense_body,
        out_shape=jax.ShapeDtypeStruct((1, 1), jnp.float32),
    )(X1, counts,
      Wenc_c, benc_c.reshape(1, _D), Bc, bbc,
      Wenc_s, benc_s.reshape(1, _D), Bs, bbs,
      Wenc_u, benc_u.reshape(1, _D), Bu, bbu,
      Wenc_g, benc_g.reshape(1, _D), Bg, bbg,
      Wro, bro.reshape(1, 128),
      Wf0, bf0.reshape(1, 512), Wf1, bf1.reshape(1, 512),
      Wf2, bf2.reshape(1, 512), Wf3, bf3.reshape(1, 512),
      Wf4, bf4.reshape(1, 512),
      Wf5, bf5.reshape(1, 1))
    return out.reshape(-1)


# speedup vs baseline: 1.2639x; 1.1861x over previous
import functools

import jax
import jax.numpy as jnp
from jax import lax
from jax.experimental import pallas as pl
from jax.experimental.pallas import tpu as pltpu
from jax.experimental.pallas import tpu_sc as plsc

_D = 256
_N = 10000
_NROWS = 80
_LANES = 16


def _counts_sc(sub_idx, bdu_idx, bdc_idx):
    mesh = plsc.VectorSubcoreMesh(core_axis_name="c", subcore_axis_name="s")

    @functools.partial(
        pl.kernel,
        out_type=jax.ShapeDtypeStruct((_NROWS, _N), jnp.float32),
        mesh=mesh,
        compiler_params=pltpu.CompilerParams(needs_layout_passes=False),
        scratch_types=[
            pltpu.VMEM((_N,), jnp.float32),
            pltpu.VMEM((_N,), jnp.float32),
            pltpu.VMEM((768,), jnp.int32),
            pltpu.SemaphoreType.DMA((3,)),
            pltpu.SemaphoreType.DMA((2,)),
        ],
    )
    def counts_kernel(sub_hbm, bdu_hbm, bdc_hbm, out_hbm,
                      buf0, buf1, idxbuf, isem, esem):
        wid = lax.axis_index("s") * 2 + lax.axis_index("c")
        ones16 = jnp.full((_LANES,), 1.0, jnp.float32)
        zeros16 = jnp.zeros((_LANES,), jnp.float32)
        lane = lax.iota(jnp.int32, _LANES)

        cp0 = pltpu.async_copy(bdc_hbm.at[wid], idxbuf.at[pl.ds(0, 128)],
                               isem.at[0])
        cp1 = pltpu.async_copy(bdc_hbm.at[wid + 32],
                               idxbuf.at[pl.ds(128, 128)], isem.at[1])
        for p in range(4):
            @pl.when(wid == 16 + p)
            def _(p=p):
                pltpu.async_copy(sub_hbm.at[pl.ds(128 * p, 128)],
                                 idxbuf.at[pl.ds(256, 128)], isem.at[2])
        for p in range(8):
            @pl.when(wid == 20 + p)
            def _(p=p):
                pltpu.async_copy(bdu_hbm.at[pl.ds(512 * p, 512)],
                                 idxbuf.at[pl.ds(256, 512)], isem.at[2])

        def fill(buf):
            def body(i, c):
                buf[pl.ds(i * _LANES, _LANES)] = jnp.zeros(
                    (_LANES,), jnp.float32)
                return c
            lax.fori_loop(0, _N // _LANES, body, 0, unroll=25)

        def scatter(buf, start, count):
            def body(i, c):
                iv = idxbuf[pl.ds(start + i * _LANES, _LANES)]
                for l in range(_LANES):
                    plsc.addupdate_scatter(buf, [iv], ones16,
                                           mask=lane == l)
                return c
            lax.fori_loop(0, count // _LANES, body, 0)

        def unscatter(buf, start, count):
            def body(i, c):
                iv = idxbuf[pl.ds(start + i * _LANES, _LANES)]
                plsc.store_scatter(buf, [iv], zeros16)
                return c
            lax.fori_loop(0, count // _LANES, body, 0)

        def emit(buf, r, slot):
            return pltpu.async_copy(buf, out_hbm.at[r], esem.at[slot])

        fill(buf0)
        fill(buf1)

        cp0.wait()
        scatter(buf0, 0, 128)
        e0 = emit(buf0, wid, 0)
        cp1.wait()
        scatter(buf1, 128, 128)
        emit(buf1, wid + 32, 1)

        for p in range(4):
            @pl.when(wid == 16 + p)
            def _(p=p):
                e0.wait()
                unscatter(buf0, 0, 128)
                pltpu.make_async_copy(sub_hbm.at[pl.ds(128 * p, 128)],
                                      idxbuf.at[pl.ds(256, 128)],
                                      isem.at[2]).wait()
                scatter(buf0, 256, 128)
                emit(buf0, 64 + p, 0)
        for p in range(8):
            @pl.when(wid == 20 + p)
            def _(p=p):
                e0.wait()
                unscatter(buf0, 0, 128)
                pltpu.make_async_copy(bdu_hbm.at[pl.ds(512 * p, 512)],
                                      idxbuf.at[pl.ds(256, 512)],
                                      isem.at[2]).wait()
                scatter(buf0, 256, 512)
                emit(buf0, 68 + p, 0)
        for p in range(4):
            @pl.when(wid == 28 + p)
            def _(p=p):
                e0.wait()
                unscatter(buf0, 0, 128)
                emit(buf0, 76 + p, 0)

        pltpu.make_async_copy(buf0, out_hbm.at[wid], esem.at[0]).wait()
        pltpu.make_async_copy(buf1, out_hbm.at[wid], esem.at[1]).wait()

    return counts_kernel(sub_idx, bdu_idx, bdc_idx)


def _mm(a, b):
    return lax.dot_general(a, b, (((1,), (0,)), ((), ())),
                           precision=lax.Precision.HIGHEST,
                           preferred_element_type=jnp.float32)


def _mmh(a, b):
    return lax.dot_general(a, b, (((1,), (0,)), ((), ())),
                           precision=lax.Precision.HIGHEST,
                           preferred_element_type=jnp.float32)


def _dense_body(x1_ref, c_ref,
                wc_ref, bc_ref, cwc_ref, cbc_ref,
                ws_ref, bs_ref, cws_ref, cbs_ref,
                wu_ref, bu_ref, cwu_ref, cbu_ref,
                wg_ref, bg_ref, cwg_ref, cbg_ref,
                wro_ref, bro_ref,
                wf0_ref, bf0_ref, wf1_ref, bf1_ref, wf2_ref, bf2_ref,
                wf3_ref, bf3_ref, wf4_ref, bf4_ref, wf5_ref, bf5_ref,
                out_ref, acc_ref, zg_ref):
    j = pl.program_id(0)
    acc_ref[:, pl.ds(j * 128, 128)] = _mmh(c_ref[...], x1_ref[...])
    zg_ref[0:1, pl.ds(j * 128, 128)] = jnp.sum(x1_ref[...], axis=0,
                                               keepdims=True)

    @pl.when(j == _D // 128 - 1)
    def _():
        _dense_tail(acc_ref[...], zg_ref[0:1, :],
                    wc_ref, bc_ref, cwc_ref, cbc_ref,
                    ws_ref, bs_ref, cws_ref, cbs_ref,
                    wu_ref, bu_ref, cwu_ref, cbu_ref,
                    wg_ref, bg_ref, cwg_ref, cbg_ref,
                    wro_ref, bro_ref,
                    wf0_ref, bf0_ref, wf1_ref, bf1_ref, wf2_ref, bf2_ref,
                    wf3_ref, bf3_ref, wf4_ref, bf4_ref, wf5_ref, bf5_ref,
                    out_ref)


def _conv_mats(cw_ref, cb_ref, R1, R2, M):
    Wc = cw_ref[...]
    WcT_R2 = lax.dot_general(Wc, R2, (((0,), (0,)), ((), ())),
                             precision=lax.Precision.HIGHEST,
                             preferred_element_type=jnp.float32)
    B = _mm(R1, WcT_R2) * M
    bb = _mm(cb_ref[...].reshape(1, 8), R2)
    return B, bb


def _dense_tail(acc, Zg,
                wc_ref, bc_ref, cwc_ref, cbc_ref,
                ws_ref, bs_ref, cws_ref, cbs_ref,
                wu_ref, bu_ref, cwu_ref, cbu_ref,
                wg_ref, bg_ref, cwg_ref, cbg_ref,
                wro_ref, bro_ref,
                wf0_ref, bf0_ref, wf1_ref, bf1_ref, wf2_ref, bf2_ref,
                wf3_ref, bf3_ref, wf4_ref, bf4_ref, wf5_ref, bf5_ref,
                out_ref):
    R1 = (lax.broadcasted_iota(jnp.int32, (_D, 16), 0) % 16
          == lax.broadcasted_iota(jnp.int32, (_D, 16), 1)
          ).astype(jnp.float32)
    R2 = (lax.broadcasted_iota(jnp.int32, (8, 128), 0)
          == lax.broadcasted_iota(jnp.int32, (8, 128), 1) // 16
          ).astype(jnp.float32)
    M = (lax.broadcasted_iota(jnp.int32, (_D, 128), 0) // 16
         == lax.broadcasted_iota(jnp.int32, (_D, 128), 1) % 16
         ).astype(jnp.float32)

    Zc = acc[0:64]
    Zx = acc[64:80]
    rows16 = lax.broadcasted_iota(jnp.int32, (16, 1), 0)
    Zs = jnp.sum(jnp.where(rows16 < 4, Zx, 0.0), axis=0, keepdims=True)
    Zu = jnp.sum(jnp.where((rows16 >= 4) & (rows16 < 12), Zx, 0.0),
                 axis=0, keepdims=True)

    def enc(Z, w_ref, b_ref, cw_ref, cb_ref):
        B, bb = _conv_mats(cw_ref, cb_ref, R1, R2, M)
        A = jnp.maximum(_mm(Z, w_ref[...]) + b_ref[...][None, :], 0.0)
        return jnp.tanh(2.0 * (_mm(A, B) + bb))

    Hc = enc(Zc, wc_ref, bc_ref, cwc_ref, cbc_ref)
    HS = enc(Zs, ws_ref, bs_ref, cws_ref, cbs_ref)
    HBDU = enc(Zu, wu_ref, bu_ref, cwu_ref, cbu_ref)
    HG = enc(Zg, wg_ref, bg_ref, cwg_ref, cbg_ref)

    S = jnp.sum(Hc, axis=0, keepdims=True)
    HBDC = jnp.maximum(_mm(S, wro_ref[...]) + bro_ref[...][None, :], 0.0)

    h = jnp.concatenate([HG, HS, HBDU, HBDC], axis=1)
    for w_ref, b_ref in ((wf0_ref, bf0_ref), (wf1_ref, bf1_ref),
                         (wf2_ref, bf2_ref), (wf3_ref, bf3_ref),
                         (wf4_ref, bf4_ref)):
        z = _mm(h, w_ref[...]) + b_ref[...][None, :]
        h = jnp.where(z > 0.0, z, jnp.exp(z) - 1.0)
    q = _mm(h, wf5_ref[...]) + bf5_ref[...][None, :]
    out_ref[...] = q


def kernel(X1, X2, sub_u_idx, sub_v_idx, bdu_u_idx, bdu_v_idx,
           bdc_u_idx, bdc_v_idx,
           Wenc_g, benc_g, Wconv_g, bconv_g,
           Wenc_s, benc_s, Wconv_s, bconv_s,
           Wenc_u, benc_u, Wconv_u, bconv_u,
           Wenc_c, benc_c, Wconv_c, bconv_c,
           Wro, bro, Wf0, bf0, Wf1, bf1, Wf2, bf2, Wf3, bf3, Wf4, bf4,
           Wf5, bf5):
    counts = _counts_sc(sub_u_idx, bdu_u_idx, bdc_u_idx)

    weights = (
        Wenc_c, benc_c, Wconv_c, bconv_c,
        Wenc_s, benc_s, Wconv_s, bconv_s,
        Wenc_u, benc_u, Wconv_u, bconv_u,
        Wenc_g, benc_g, Wconv_g, bconv_g,
        Wro, bro,
        Wf0, bf0, Wf1, bf1, Wf2, bf2, Wf3, bf3, Wf4, bf4, Wf5, bf5)

    def _full(a):
        return pl.BlockSpec(a.shape, lambda j: (0,) * a.ndim)

    out = pl.pallas_call(
        _dense_body,
        grid=(_D // 128,),
        in_specs=[
            pl.BlockSpec((_N, 128), lambda j: (0, j)),
            pl.BlockSpec((_NROWS, _N), lambda j: (0, 0)),
        ] + [_full(w) for w in weights],
        out_specs=pl.BlockSpec((1, 1), lambda j: (0, 0)),
        out_shape=jax.ShapeDtypeStruct((1, 1), jnp.float32),
        scratch_shapes=[
            pltpu.VMEM((_NROWS, _D), jnp.float32),
            pltpu.VMEM((8, _D), jnp.float32),
        ],
    )(X1, counts, *weights)
    return out.reshape(-1)
